# baseline (speedup 1.0000x reference)
import functools

import jax
import jax.numpy as jnp
from jax import lax
from jax.experimental import pallas as pl
from jax.experimental.pallas import tpu as pltpu
from jax.experimental.pallas import tpu_sc as plsc

N = 10000
E = 320000
IN_DIM = 128
H = 32
OUT_DIM = 128
R = 4

GROUP = 128
NW = 32
G = 2560
EP = G * GROUP
GPT0 = 104
GPT1 = 56
GPTMAX = max(GPT0, GPT1)
NBUF = 2
NP = 10240
ROWS_PER_TILE = NP // 16
ZCH = 128
NT = N * R
SW = 16


BN = 2000


def _embed_body(x_ref, wf_ref, bf_ref, ws_ref, wd_ref, brf_ref,
                h0_ref, a_ref, b_ref):
    h0 = jnp.dot(x_ref[...], wf_ref[...], preferred_element_type=jnp.float32)
    h0 = h0 + bf_ref[...]
    h0_ref[...] = h0
    a_ref[...] = jnp.dot(h0, ws_ref[...], preferred_element_type=jnp.float32)
    b_ref[...] = jnp.dot(h0, wd_ref[...],
                         preferred_element_type=jnp.float32) + brf_ref[...]


def _embed(x, wf, bf, ws, wd, brf):
    grid = N // BN
    return pl.pallas_call(
        _embed_body,
        grid=(grid,),
        in_specs=[
            pl.BlockSpec((BN, IN_DIM), lambda i: (i, 0)),
            pl.BlockSpec((IN_DIM, H), lambda i: (0, 0)),
            pl.BlockSpec((1, H), lambda i: (0, 0)),
            pl.BlockSpec((H, R * H), lambda i: (0, 0)),
            pl.BlockSpec((H, R * H), lambda i: (0, 0)),
            pl.BlockSpec((1, R * H), lambda i: (0, 0)),
        ],
        out_specs=(
            pl.BlockSpec((BN, H), lambda i: (i, 0)),
            pl.BlockSpec((BN, R * H), lambda i: (i, 0)),
            pl.BlockSpec((BN, R * H), lambda i: (i, 0)),
        ),
        out_shape=(
            jax.ShapeDtypeStruct((N, H), jnp.float32),
            jax.ShapeDtypeStruct((N, R * H), jnp.float32),
            jax.ShapeDtypeStruct((N, R * H), jnp.float32),
        ),
    )(x, wf, bf, ws, wd, brf)


def _sb(s4, b_ref):
    acc = s4[:, 0:1] * b_ref[:, 0 * H:1 * H]
    for r in range(1, R):
        acc = acc + s4[:, r:r + 1] * b_ref[:, r * H:(r + 1) * H]
    return acc


def _mid_body(p_ref, s_ref, b1_ref, ws_ref, wd_ref, brf_ref,
              h_ref, a_ref, b_ref):
    s4 = s_ref[0] + s_ref[1]
    h = p_ref[0] + p_ref[1] + _sb(s4, b1_ref)
    h_ref[...] = h
    a_ref[...] = jnp.dot(h, ws_ref[...], preferred_element_type=jnp.float32)
    b_ref[...] = jnp.dot(h, wd_ref[...],
                         preferred_element_type=jnp.float32) + brf_ref[...]


def _mid(p, s, b1, ws, wd, brf):
    grid = N // BN
    return pl.pallas_call(
        _mid_body,
        grid=(grid,),
        in_specs=[
            pl.BlockSpec((2, BN, H), lambda i: (0, i, 0)),
            pl.BlockSpec((2, BN, SW), lambda i: (0, i, 0)),
            pl.BlockSpec((BN, R * H), lambda i: (i, 0)),
            pl.BlockSpec((H, R * H), lambda i: (0, 0)),
            pl.BlockSpec((H, R * H), lambda i: (0, 0)),
            pl.BlockSpec((1, R * H), lambda i: (0, 0)),
        ],
        out_specs=(
            pl.BlockSpec((BN, H), lambda i: (i, 0)),
            pl.BlockSpec((BN, R * H), lambda i: (i, 0)),
            pl.BlockSpec((BN, R * H), lambda i: (i, 0)),
        ),
        out_shape=(
            jax.ShapeDtypeStruct((N, H), jnp.float32),
            jax.ShapeDtypeStruct((N, R * H), jnp.float32),
            jax.ShapeDtypeStruct((N, R * H), jnp.float32),
        ),
    )(p, s, b1, ws, wd, brf)


def _lrelu(t):
    return jnp.where(t > 0, t, 0.01 * t)


def _final_body(p2_ref, s_ref, b2_ref, h1_ref, h0_ref, wo0_ref, bo0_ref,
                wo1_ref, bo1_ref, wo2_ref, bo2_ref, out_ref):
    s4 = s_ref[0] + s_ref[1]
    h2 = p2_ref[0] + p2_ref[1] + _sb(s4, b2_ref)
    t2 = jnp.dot(h2, wo2_ref[...], preferred_element_type=jnp.float32) + bo2_ref[...]
    t1 = jnp.dot(h1_ref[...], wo1_ref[...],
                 preferred_element_type=jnp.float32) + bo1_ref[...]
    t0 = jnp.dot(h0_ref[...], wo0_ref[...],
                 preferred_element_type=jnp.float32) + bo0_ref[...]
    out_ref[...] = _lrelu(t2) + _lrelu(t1) + _lrelu(t0)


def _final(p2, s, b2, h1, h0, wo0, bo0, wo1, bo1, wo2, bo2):
    grid = N // BN
    wspec = pl.BlockSpec((H, OUT_DIM), lambda i: (0, 0))
    bspec = pl.BlockSpec((1, OUT_DIM), lambda i: (0, 0))
    return pl.pallas_call(
        _final_body,
        grid=(grid,),
        in_specs=[
            pl.BlockSpec((2, BN, H), lambda i: (0, i, 0)),
            pl.BlockSpec((2, BN, SW), lambda i: (0, i, 0)),
            pl.BlockSpec((BN, R * H), lambda i: (i, 0)),
            pl.BlockSpec((BN, H), lambda i: (i, 0)),
            pl.BlockSpec((BN, H), lambda i: (i, 0)),
            wspec, bspec, wspec, bspec, wspec, bspec,
        ],
        out_specs=pl.BlockSpec((BN, OUT_DIM), lambda i: (i, 0)),
        out_shape=jax.ShapeDtypeStruct((N, OUT_DIM), jnp.float32),
    )(p2, s, b2, h1, h0, wo0, bo0, wo1, bo1, wo2, bo2)


def _sc_layer_body_factory(with_s):
    def body(*refs):
        if with_s:
            (a_hbm, et_hbm, src_hbm, dst_hbm, typ_hbm, lb_hbm, out_hbm,
             sout_hbm) = refs[:8]
            refs = refs[8:]
            idxa_v, dst_v, typ_v, w_v, lb_v = refs[:5]
            refs = refs[5:]
            abufs = refs[:NBUF]
            obufs = refs[NBUF:2 * NBUF]
            sbufs = refs[2 * NBUF:3 * NBUF]
            zbuf, zbuf16, acc, sacc = refs[3 * NBUF:3 * NBUF + 4]
            sems = refs[3 * NBUF + 4:]
            sgas = sems[:NBUF]
            sss = sems[NBUF:2 * NBUF]
            ssss = sems[2 * NBUF:3 * NBUF]
        else:
            (a_hbm, et_hbm, src_hbm, dst_hbm, typ_hbm, lb_hbm,
             out_hbm) = refs[:7]
            refs = refs[7:]
            idxa_v, dst_v, typ_v, w_v, lb_v = refs[:5]
            refs = refs[5:]
            abufs = refs[:NBUF]
            obufs = refs[NBUF:2 * NBUF]
            zbuf, acc = refs[2 * NBUF:2 * NBUF + 2]
            sems = refs[2 * NBUF + 2:]
            sgas = sems[:NBUF]
            sss = sems[NBUF:2 * NBUF]

        cid = lax.axis_index("c")
        sid = lax.axis_index("s")
        ng = jnp.where(cid == 0, GPT0, GPT1)
        gbase = jnp.where(cid == 0, sid * GPT0, 16 * GPT0 + sid * GPT1)
        sbase = jnp.minimum(gbase, G - GPTMAX)
        off = gbase - sbase
        pltpu.sync_copy(et_hbm.at[pl.ds(sbase, GPTMAX)], w_v)
        pltpu.sync_copy(src_hbm.at[pl.ds(sbase, GPTMAX)], idxa_v)
        pltpu.sync_copy(dst_hbm.at[pl.ds(sbase, GPTMAX)], dst_v)
        pltpu.sync_copy(typ_hbm.at[pl.ds(sbase, GPTMAX)], typ_v)
        pltpu.sync_copy(lb_hbm, lb_v)
        lb16 = lb_v[0:16]
        lam = lb16[0]
        beta = lb16[1]
        nvalid = E // GROUP

        def wprep(r, c):
            gid = sbase + r
            for j in range(GROUP // 16):
                sl = pl.ds(j * 16, 16)
                et16 = w_v[r, sl]
                w16 = lam * jnp.exp(-beta * jnp.abs(et16))
                w_v[r, sl] = jnp.where(gid < nvalid, w16,
                                       jnp.zeros((16,), jnp.float32))
                ia16 = idxa_v[r, sl] * R + typ_v[r, sl]
                idxa_v[r, sl] = ia16
            return c

        lax.fori_loop(0, GPTMAX, wprep, 0)

        def zb(i, c):
            zbuf[i, 0:16] = jnp.zeros((16,), jnp.float32)
            zbuf[i, 16:32] = jnp.zeros((16,), jnp.float32)
            if with_s:
                zbuf16[i, 0:16] = jnp.zeros((16,), jnp.float32)
            return c

        lax.fori_loop(0, ZCH, zb, 0)
        rbase = sid * ROWS_PER_TILE
        for j in range(ROWS_PER_TILE // ZCH):
            pltpu.sync_copy(zbuf.at[pl.ds(0, ZCH)],
                            acc.at[pl.ds(rbase + j * ZCH, ZCH)])
            if with_s:
                pltpu.sync_copy(zbuf16.at[pl.ds(0, ZCH)],
                                sacc.at[pl.ds(rbase + j * ZCH, ZCH)])
        plsc.subcore_barrier()

        for p in range(NBUF):
            pltpu.async_copy(a_hbm.at[idxa_v.at[off + p]], abufs[p], sgas[p])

        iota16 = lax.iota(jnp.int32, 16)

        def ring(kk, c):
            for p in range(NBUF):
                k = kk * NBUF + p
                ab, ob = abufs[p], obufs[p]
                pltpu.make_async_copy(a_hbm.at[idxa_v.at[off + k]], ab,
                                      sgas[p]).wait()

                @pl.when(kk > 0)
                def _():
                    pltpu.make_async_copy(ob, acc.at[dst_v.at[off + k]],
                                          sss[p]).wait()
                    if with_s:
                        pltpu.make_async_copy(sbufs[p],
                                              sacc.at[dst_v.at[off + k]],
                                              ssss[p]).wait()

                def ebody(j, cc):
                    wv16 = w_v[off + k, pl.ds(j * 16, 16)]
                    if with_s:
                        tv16 = typ_v[off + k, pl.ds(j * 16, 16)]
                    for ll in range(16):
                        i = j * 16 + ll
                        wv = wv16[ll]
                        ob[i, 0:16] = ab[i, 0:16] * wv
                        ob[i, 16:32] = ab[i, 16:32] * wv
                        if with_s:
                            sbufs[p][i, 0:16] = jnp.where(
                                iota16 == tv16[ll], wv, 0.0)
                    return cc

                lax.fori_loop(0, GROUP // 16, ebody, 0)

                @pl.when(k + NBUF < ng)
                def _():
                    pltpu.async_copy(a_hbm.at[idxa_v.at[off + k + NBUF]], ab,
                                     sgas[p])

                pltpu.async_copy(ob, acc.at[dst_v.at[off + k]], sss[p],
                                 add=True)
                if with_s:
                    pltpu.async_copy(sbufs[p], sacc.at[dst_v.at[off + k]],
                                     ssss[p], add=True)
            return c

        lax.fori_loop(0, ng // NBUF, ring, 0)
        for p in range(NBUF):
            pltpu.make_async_copy(obufs[p],
                                  acc.at[dst_v.at[off + ng - NBUF + p]],
                                  sss[p]).wait()
            if with_s:
                pltpu.make_async_copy(sbufs[p],
                                      sacc.at[dst_v.at[off + ng - NBUF + p]],
                                      ssss[p]).wait()
        plsc.subcore_barrier()

        pltpu.sync_copy(acc.at[pl.ds(rbase, ROWS_PER_TILE)],
                        out_hbm.at[cid, pl.ds(rbase, ROWS_PER_TILE)])
        if with_s:
            pltpu.sync_copy(sacc.at[pl.ds(rbase, ROWS_PER_TILE)],
                            sout_hbm.at[cid, pl.ds(rbase, ROWS_PER_TILE)])

    return body


def _sc_layer1(a2d, et2, src2, dst2, typ2, lb):
    mesh = plsc.VectorSubcoreMesh(core_axis_name="c", subcore_axis_name="s")
    scratch = [
        pltpu.VMEM((GPTMAX, GROUP), jnp.int32),
        pltpu.VMEM((GPTMAX, GROUP), jnp.int32),
        pltpu.VMEM((GPTMAX, GROUP), jnp.int32),
        pltpu.VMEM((GPTMAX, GROUP), jnp.float32),
        pltpu.VMEM((16,), jnp.float32),
    ]
    scratch += [pltpu.VMEM((GROUP, H), jnp.float32)] * (2 * NBUF)
    scratch += [pltpu.VMEM((GROUP, SW), jnp.float32)] * NBUF
    scratch += [
        pltpu.VMEM((ZCH, H), jnp.float32),
        pltpu.VMEM((ZCH, SW), jnp.float32),
        pltpu.VMEM_SHARED((NP, H), jnp.float32),
        pltpu.VMEM_SHARED((NP, SW), jnp.float32),
    ]
    scratch += [pltpu.SemaphoreType.DMA] * (3 * NBUF)
    kern = functools.partial(
        pl.kernel,
        mesh=mesh,
        compiler_params=pltpu.CompilerParams(use_tc_tiling_on_sc=False),
        out_type=(
            jax.ShapeDtypeStruct((2, NP, H), jnp.float32),
            jax.ShapeDtypeStruct((2, NP, SW), jnp.float32),
        ),
        scratch_types=scratch,
    )(_sc_layer_body_factory(True))
    return kern(a2d, et2, src2, dst2, typ2, lb)


def _sc_layer2(a2d, et2, src2, dst2, typ2, lb):
    mesh = plsc.VectorSubcoreMesh(core_axis_name="c", subcore_axis_name="s")
    scratch = [
        pltpu.VMEM((GPTMAX, GROUP), jnp.int32),
        pltpu.VMEM((GPTMAX, GROUP), jnp.int32),
        pltpu.VMEM((GPTMAX, GROUP), jnp.int32),
        pltpu.VMEM((GPTMAX, GROUP), jnp.float32),
        pltpu.VMEM((16,), jnp.float32),
    ]
    scratch += [pltpu.VMEM((GROUP, H), jnp.float32)] * (2 * NBUF)
    scratch += [
        pltpu.VMEM((ZCH, H), jnp.float32),
        pltpu.VMEM_SHARED((NP, H), jnp.float32),
    ]
    scratch += [pltpu.SemaphoreType.DMA] * (2 * NBUF)
    kern = functools.partial(
        pl.kernel,
        mesh=mesh,
        compiler_params=pltpu.CompilerParams(use_tc_tiling_on_sc=False),
        out_type=jax.ShapeDtypeStruct((2, NP, H), jnp.float32),
        scratch_types=scratch,
    )(_sc_layer_body_factory(False))
    return kern(a2d, et2, src2, dst2, typ2, lb)


def kernel(x, edge_time, lambda_sym, beta, Wf, bf, Wr1, br1, Wr2, br2,
           Wo0, bo0, Wo1, bo1, Wo2, bo2, edge_index, edge_type):
    ws1 = jnp.transpose(Wr1[:, :H, :], (1, 0, 2)).reshape(H, R * H)
    wd1 = jnp.transpose(Wr1[:, H:, :], (1, 0, 2)).reshape(H, R * H)
    ws2 = jnp.transpose(Wr2[:, :H, :], (1, 0, 2)).reshape(H, R * H)
    wd2 = jnp.transpose(Wr2[:, H:, :], (1, 0, 2)).reshape(H, R * H)
    brf1 = br1.reshape(1, R * H)
    brf2 = br2.reshape(1, R * H)
    bfr = bf.reshape(1, H)
    bo0r = bo0.reshape(1, OUT_DIM)
    bo1r = bo1.reshape(1, OUT_DIM)
    bo2r = bo2.reshape(1, OUT_DIM)

    pad = EP - E
    et2 = jnp.pad(edge_time, (0, pad)).reshape(G, GROUP)
    src2 = jnp.pad(edge_index[0], (0, pad)).reshape(G, GROUP)
    dst2 = jnp.pad(edge_index[1], (0, pad)).reshape(G, GROUP)
    typ2 = jnp.pad(edge_type, (0, pad)).reshape(G, GROUP)

    lb = jnp.concatenate(
        [lambda_sym.reshape(1), beta.reshape(1),
         jnp.zeros((14,), jnp.float32)])

    h0, a1, b1 = _embed(x, Wf, bfr, ws1, wd1, brf1)
    p1, s1 = _sc_layer1(a1.reshape(NT, H), et2, src2, dst2, typ2, lb)
    p1 = p1[:, :N, :]
    s = s1[:, :N, :]
    h1, a2, b2 = _mid(p1, s, b1, ws2, wd2, brf2)
    p2 = _sc_layer2(a2.reshape(NT, H), et2, src2, dst2, typ2, lb)[:, :N, :]
    out = _final(p2, s, b2, h1, h0, Wo0, bo0r, Wo1, bo1r, Wo2, bo2r)
    return out


# speedup vs baseline: 1.0412x; 1.0412x over previous
import functools

import jax
import jax.numpy as jnp
from jax import lax
from jax.experimental import pallas as pl
from jax.experimental.pallas import tpu as pltpu
from jax.experimental.pallas import tpu_sc as plsc

N = 10000
E = 320000
IN_DIM = 128
H = 32
OUT_DIM = 128
R = 4

GROUP = 128
NW = 32
G = 2560
EP = G * GROUP
GPT0 = 120
GPT1 = 40
GPTMAX = max(GPT0, GPT1)
NP = 10240
ROWS_PER_TILE = NP // 16
ZCH = 128
NT = N * R
SW = 16


def _edge_prep_body(et_ref, src_ref, typ_ref, lam_ref, beta_ref,
                    w_ref, ia_ref):
    lam = lam_ref[0, 0]
    beta = beta_ref[0, 0]
    valid = lax.broadcasted_iota(jnp.int32, (G, GROUP), 0) < (E // GROUP)
    w = lam * jnp.exp(-beta * jnp.abs(et_ref[...]))
    w_ref[...] = jnp.where(valid, w, 0.0)
    ia_ref[...] = src_ref[...] * R + typ_ref[...]


def _edge_prep(et2, src2, typ2, lam, beta):
    return pl.pallas_call(
        _edge_prep_body,
        out_shape=(
            jax.ShapeDtypeStruct((G, GROUP), jnp.float32),
            jax.ShapeDtypeStruct((G, GROUP), jnp.int32),
        ),
    )(et2, src2, typ2, lam, beta)


BN = 2000


def _embed_body(x_ref, wf_ref, bf_ref, ws_ref, wd_ref, brf_ref,
                h0_ref, a_ref, b_ref):
    h0 = jnp.dot(x_ref[...], wf_ref[...], preferred_element_type=jnp.float32)
    h0 = h0 + bf_ref[...]
    h0_ref[...] = h0
    a_ref[...] = jnp.dot(h0, ws_ref[...], preferred_element_type=jnp.float32)
    b_ref[...] = jnp.dot(h0, wd_ref[...],
                         preferred_element_type=jnp.float32) + brf_ref[...]


def _embed(x, wf, bf, ws, wd, brf):
    grid = N // BN
    return pl.pallas_call(
        _embed_body,
        grid=(grid,),
        in_specs=[
            pl.BlockSpec((BN, IN_DIM), lambda i: (i, 0)),
            pl.BlockSpec((IN_DIM, H), lambda i: (0, 0)),
            pl.BlockSpec((1, H), lambda i: (0, 0)),
            pl.BlockSpec((H, R * H), lambda i: (0, 0)),
            pl.BlockSpec((H, R * H), lambda i: (0, 0)),
            pl.BlockSpec((1, R * H), lambda i: (0, 0)),
        ],
        out_specs=(
            pl.BlockSpec((BN, H), lambda i: (i, 0)),
            pl.BlockSpec((BN, R * H), lambda i: (i, 0)),
            pl.BlockSpec((BN, R * H), lambda i: (i, 0)),
        ),
        out_shape=(
            jax.ShapeDtypeStruct((N, H), jnp.float32),
            jax.ShapeDtypeStruct((N, R * H), jnp.float32),
            jax.ShapeDtypeStruct((N, R * H), jnp.float32),
        ),
    )(x, wf, bf, ws, wd, brf)


def _sb(s4, b_ref):
    acc = s4[:, 0:1] * b_ref[:, 0 * H:1 * H]
    for r in range(1, R):
        acc = acc + s4[:, r:r + 1] * b_ref[:, r * H:(r + 1) * H]
    return acc


def _mid_body(p_ref, s_ref, b1_ref, ws_ref, wd_ref, brf_ref,
              h_ref, a_ref, b_ref):
    s4 = s_ref[0] + s_ref[1]
    h = p_ref[0] + p_ref[1] + _sb(s4, b1_ref)
    h_ref[...] = h
    a_ref[...] = jnp.dot(h, ws_ref[...], preferred_element_type=jnp.float32)
    b_ref[...] = jnp.dot(h, wd_ref[...],
                         preferred_element_type=jnp.float32) + brf_ref[...]


def _mid(p, s, b1, ws, wd, brf):
    grid = N // BN
    return pl.pallas_call(
        _mid_body,
        grid=(grid,),
        in_specs=[
            pl.BlockSpec((2, BN, H), lambda i: (0, i, 0)),
            pl.BlockSpec((2, BN, SW), lambda i: (0, i, 0)),
            pl.BlockSpec((BN, R * H), lambda i: (i, 0)),
            pl.BlockSpec((H, R * H), lambda i: (0, 0)),
            pl.BlockSpec((H, R * H), lambda i: (0, 0)),
            pl.BlockSpec((1, R * H), lambda i: (0, 0)),
        ],
        out_specs=(
            pl.BlockSpec((BN, H), lambda i: (i, 0)),
            pl.BlockSpec((BN, R * H), lambda i: (i, 0)),
            pl.BlockSpec((BN, R * H), lambda i: (i, 0)),
        ),
        out_shape=(
            jax.ShapeDtypeStruct((N, H), jnp.float32),
            jax.ShapeDtypeStruct((N, R * H), jnp.float32),
            jax.ShapeDtypeStruct((N, R * H), jnp.float32),
        ),
    )(p, s, b1, ws, wd, brf)


def _lrelu(t):
    return jnp.where(t > 0, t, 0.01 * t)


def _final_body(p2_ref, s_ref, b2_ref, h1_ref, h0_ref, wo0_ref, bo0_ref,
                wo1_ref, bo1_ref, wo2_ref, bo2_ref, out_ref):
    s4 = s_ref[0] + s_ref[1]
    h2 = p2_ref[0] + p2_ref[1] + _sb(s4, b2_ref)
    t2 = jnp.dot(h2, wo2_ref[...], preferred_element_type=jnp.float32) + bo2_ref[...]
    t1 = jnp.dot(h1_ref[...], wo1_ref[...],
                 preferred_element_type=jnp.float32) + bo1_ref[...]
    t0 = jnp.dot(h0_ref[...], wo0_ref[...],
                 preferred_element_type=jnp.float32) + bo0_ref[...]
    out_ref[...] = _lrelu(t2) + _lrelu(t1) + _lrelu(t0)


def _final(p2, s, b2, h1, h0, wo0, bo0, wo1, bo1, wo2, bo2):
    grid = N // BN
    wspec = pl.BlockSpec((H, OUT_DIM), lambda i: (0, 0))
    bspec = pl.BlockSpec((1, OUT_DIM), lambda i: (0, 0))
    return pl.pallas_call(
        _final_body,
        grid=(grid,),
        in_specs=[
            pl.BlockSpec((2, BN, H), lambda i: (0, i, 0)),
            pl.BlockSpec((2, BN, SW), lambda i: (0, i, 0)),
            pl.BlockSpec((BN, R * H), lambda i: (i, 0)),
            pl.BlockSpec((BN, H), lambda i: (i, 0)),
            pl.BlockSpec((BN, H), lambda i: (i, 0)),
            wspec, bspec, wspec, bspec, wspec, bspec,
        ],
        out_specs=pl.BlockSpec((BN, OUT_DIM), lambda i: (i, 0)),
        out_shape=jax.ShapeDtypeStruct((N, OUT_DIM), jnp.float32),
    )(p2, s, b2, h1, h0, wo0, bo0, wo1, bo1, wo2, bo2)


def _sc_layer_body_factory(with_s):
    def body(*refs):
        if with_s:
            (a_hbm, idxa_hbm, dst_hbm, typ_hbm, w_hbm, out_hbm, sout_hbm,
             idxa_v, dst_v, typ_v, w_v, a0, a1, o0, o1, s0b, s1b,
             zbuf, zbuf16, acc, sacc,
             sga0, sga1, ss0, ss1, sss0, sss1) = refs
        else:
            (a_hbm, idxa_hbm, dst_hbm, w_hbm, out_hbm,
             idxa_v, dst_v, w_v, a0, a1, o0, o1,
             zbuf, acc,
             sga0, sga1, ss0, ss1) = refs

        cid = lax.axis_index("c")
        sid = lax.axis_index("s")
        ng = jnp.where(cid == 0, GPT0, GPT1)
        gbase = jnp.where(cid == 0, sid * GPT0, 16 * GPT0 + sid * GPT1)
        sbase = jnp.minimum(gbase, G - GPTMAX)
        off = gbase - sbase
        pltpu.sync_copy(idxa_hbm.at[pl.ds(sbase, GPTMAX)], idxa_v)
        pltpu.sync_copy(dst_hbm.at[pl.ds(sbase, GPTMAX)], dst_v)
        pltpu.sync_copy(w_hbm.at[pl.ds(sbase, GPTMAX)], w_v)
        if with_s:
            pltpu.sync_copy(typ_hbm.at[pl.ds(sbase, GPTMAX)], typ_v)

        def zb(i, c):
            zbuf[i, 0:16] = jnp.zeros((16,), jnp.float32)
            zbuf[i, 16:32] = jnp.zeros((16,), jnp.float32)
            if with_s:
                zbuf16[i, 0:16] = jnp.zeros((16,), jnp.float32)
            return c

        lax.fori_loop(0, ZCH, zb, 0)
        rbase = sid * ROWS_PER_TILE
        for j in range(ROWS_PER_TILE // ZCH):
            pltpu.sync_copy(zbuf.at[pl.ds(0, ZCH)],
                            acc.at[pl.ds(rbase + j * ZCH, ZCH)])
            if with_s:
                pltpu.sync_copy(zbuf16.at[pl.ds(0, ZCH)],
                                sacc.at[pl.ds(rbase + j * ZCH, ZCH)])
        plsc.subcore_barrier()

        abufs = (a0, a1)
        obufs = (o0, o1)
        sgas = (sga0, sga1)
        sss = (ss0, ss1)
        if with_s:
            sbufs = (s0b, s1b)
            ssss = (sss0, sss1)

        for p in range(2):
            pltpu.async_copy(a_hbm.at[idxa_v.at[off + p]], abufs[p], sgas[p])

        iota16 = lax.iota(jnp.int32, 16)

        def pair(k2, c):
            for p in range(2):
                k = k2 * 2 + p
                ab, ob = abufs[p], obufs[p]
                pltpu.make_async_copy(a_hbm.at[idxa_v.at[off + k]], ab,
                                      sgas[p]).wait()

                @pl.when(k2 > 0)
                def _():
                    pltpu.make_async_copy(ob, acc.at[dst_v.at[off + k]],
                                          sss[p]).wait()
                    if with_s:
                        pltpu.make_async_copy(sbufs[p],
                                              sacc.at[dst_v.at[off + k]],
                                              ssss[p]).wait()

                def ebody(j, cc):
                    wv16 = w_v[off + k, pl.ds(j * 16, 16)]
                    if with_s:
                        tv16 = typ_v[off + k, pl.ds(j * 16, 16)]
                    for ll in range(16):
                        i = j * 16 + ll
                        wv = wv16[ll]
                        ob[i, 0:16] = ab[i, 0:16] * wv
                        ob[i, 16:32] = ab[i, 16:32] * wv
                        if with_s:
                            sbufs[p][i, 0:16] = jnp.where(
                                iota16 == tv16[ll], wv, 0.0)
                    return cc

                lax.fori_loop(0, GROUP // 16, ebody, 0)

                @pl.when(k + 2 < ng)
                def _():
                    pltpu.async_copy(a_hbm.at[idxa_v.at[off + k + 2]], ab,
                                     sgas[p])

                pltpu.async_copy(ob, acc.at[dst_v.at[off + k]], sss[p],
                                 add=True)
                if with_s:
                    pltpu.async_copy(sbufs[p], sacc.at[dst_v.at[off + k]],
                                     ssss[p], add=True)
            return c

        lax.fori_loop(0, (ng + 1) // 2, pair, 0)
        for p in range(2):
            pltpu.make_async_copy(obufs[p], acc.at[dst_v.at[off + ng - 2 + p]],
                                  sss[p]).wait()
            if with_s:
                pltpu.make_async_copy(sbufs[p],
                                      sacc.at[dst_v.at[off + ng - 2 + p]],
                                      ssss[p]).wait()
        plsc.subcore_barrier()

        pltpu.sync_copy(acc.at[pl.ds(rbase, ROWS_PER_TILE)],
                        out_hbm.at[cid, pl.ds(rbase, ROWS_PER_TILE)])
        if with_s:
            pltpu.sync_copy(sacc.at[pl.ds(rbase, ROWS_PER_TILE)],
                            sout_hbm.at[cid, pl.ds(rbase, ROWS_PER_TILE)])

    return body


def _sc_layer1(a2d, idxa2, dst2, typ2, w2):
    mesh = plsc.VectorSubcoreMesh(core_axis_name="c", subcore_axis_name="s")
    kern = functools.partial(
        pl.kernel,
        mesh=mesh,
        compiler_params=pltpu.CompilerParams(use_tc_tiling_on_sc=False),
        out_type=(
            jax.ShapeDtypeStruct((2, NP, H), jnp.float32),
            jax.ShapeDtypeStruct((2, NP, SW), jnp.float32),
        ),
        scratch_types=[
            pltpu.VMEM((GPTMAX, GROUP), jnp.int32),
            pltpu.VMEM((GPTMAX, GROUP), jnp.int32),
            pltpu.VMEM((GPTMAX, GROUP), jnp.int32),
            pltpu.VMEM((GPTMAX, GROUP), jnp.float32),
            pltpu.VMEM((GROUP, H), jnp.float32),
            pltpu.VMEM((GROUP, H), jnp.float32),
            pltpu.VMEM((GROUP, H), jnp.float32),
            pltpu.VMEM((GROUP, H), jnp.float32),
            pltpu.VMEM((GROUP, SW), jnp.float32),
            pltpu.VMEM((GROUP, SW), jnp.float32),
            pltpu.VMEM((ZCH, H), jnp.float32),
            pltpu.VMEM((ZCH, SW), jnp.float32),
            pltpu.VMEM_SHARED((NP, H), jnp.float32),
            pltpu.VMEM_SHARED((NP, SW), jnp.float32),
            pltpu.SemaphoreType.DMA,
            pltpu.SemaphoreType.DMA,
            pltpu.SemaphoreType.DMA,
            pltpu.SemaphoreType.DMA,
            pltpu.SemaphoreType.DMA,
            pltpu.SemaphoreType.DMA,
        ],
    )(_sc_layer_body_factory(True))
    return kern(a2d, idxa2, dst2, typ2, w2)


def _sc_layer2(a2d, idxa2, dst2, w2):
    mesh = plsc.VectorSubcoreMesh(core_axis_name="c", subcore_axis_name="s")
    kern = functools.partial(
        pl.kernel,
        mesh=mesh,
        compiler_params=pltpu.CompilerParams(use_tc_tiling_on_sc=False),
        out_type=jax.ShapeDtypeStruct((2, NP, H), jnp.float32),
        scratch_types=[
            pltpu.VMEM((GPTMAX, GROUP), jnp.int32),
            pltpu.VMEM((GPTMAX, GROUP), jnp.int32),
            pltpu.VMEM((GPTMAX, GROUP), jnp.float32),
            pltpu.VMEM((GROUP, H), jnp.float32),
            pltpu.VMEM((GROUP, H), jnp.float32),
            pltpu.VMEM((GROUP, H), jnp.float32),
            pltpu.VMEM((GROUP, H), jnp.float32),
            pltpu.VMEM((ZCH, H), jnp.float32),
            pltpu.VMEM_SHARED((NP, H), jnp.float32),
            pltpu.SemaphoreType.DMA,
            pltpu.SemaphoreType.DMA,
            pltpu.SemaphoreType.DMA,
            pltpu.SemaphoreType.DMA,
        ],
    )(_sc_layer_body_factory(False))
    return kern(a2d, idxa2, dst2, w2)


def kernel(x, edge_time, lambda_sym, beta, Wf, bf, Wr1, br1, Wr2, br2,
           Wo0, bo0, Wo1, bo1, Wo2, bo2, edge_index, edge_type):
    ws1 = jnp.transpose(Wr1[:, :H, :], (1, 0, 2)).reshape(H, R * H)
    wd1 = jnp.transpose(Wr1[:, H:, :], (1, 0, 2)).reshape(H, R * H)
    ws2 = jnp.transpose(Wr2[:, :H, :], (1, 0, 2)).reshape(H, R * H)
    wd2 = jnp.transpose(Wr2[:, H:, :], (1, 0, 2)).reshape(H, R * H)
    brf1 = br1.reshape(1, R * H)
    brf2 = br2.reshape(1, R * H)
    bfr = bf.reshape(1, H)
    bo0r = bo0.reshape(1, OUT_DIM)
    bo1r = bo1.reshape(1, OUT_DIM)
    bo2r = bo2.reshape(1, OUT_DIM)

    pad = EP - E
    et2 = jnp.pad(edge_time, (0, pad)).reshape(G, GROUP)
    src2 = jnp.pad(edge_index[0], (0, pad)).reshape(G, GROUP)
    dst2 = jnp.pad(edge_index[1], (0, pad)).reshape(G, GROUP)
    typ2 = jnp.pad(edge_type, (0, pad)).reshape(G, GROUP)

    w2, idxa2 = _edge_prep(et2, src2, typ2, lambda_sym, beta)

    h0, a1, b1 = _embed(x, Wf, bfr, ws1, wd1, brf1)
    p1, s1 = _sc_layer1(a1.reshape(NT, H), idxa2, dst2, typ2, w2)
    p1 = p1[:, :N, :]
    s = s1[:, :N, :]
    h1, a2, b2 = _mid(p1, s, b1, ws2, wd2, brf2)
    p2 = _sc_layer2(a2.reshape(NT, H), idxa2, dst2, w2)[:, :N, :]
    out = _final(p2, s, b2, h1, h0, Wo0, bo0r, Wo1, bo1r, Wo2, bo2r)
    return out


# speedup vs baseline: 1.0767x; 1.0341x over previous
import functools

import jax
import jax.numpy as jnp
from jax import lax
from jax.experimental import pallas as pl
from jax.experimental.pallas import tpu as pltpu
from jax.experimental.pallas import tpu_sc as plsc

N = 10000
E = 320000
IN_DIM = 128
H = 32
OUT_DIM = 128
R = 4

GROUP = 128
NW = 32
G = 2560
EP = G * GROUP
GPT0 = 112
GPT1 = 48
GPTMAX = max(GPT0, GPT1)
NP = 10240
ROWS_PER_TILE = NP // 16
ZCH = 128
NT = N * R
SW = 16


def _edge_prep_body(et_ref, src_ref, typ_ref, lam_ref, beta_ref,
                    w_ref, ia_ref):
    lam = lam_ref[0, 0]
    beta = beta_ref[0, 0]
    valid = lax.broadcasted_iota(jnp.int32, (G, GROUP), 0) < (E // GROUP)
    w = lam * jnp.exp(-beta * jnp.abs(et_ref[...]))
    w_ref[...] = jnp.where(valid, w, 0.0)
    ia_ref[...] = src_ref[...] * R + typ_ref[...]


def _edge_prep(et2, src2, typ2, lam, beta):
    return pl.pallas_call(
        _edge_prep_body,
        out_shape=(
            jax.ShapeDtypeStruct((G, GROUP), jnp.float32),
            jax.ShapeDtypeStruct((G, GROUP), jnp.int32),
        ),
    )(et2, src2, typ2, lam, beta)


BN = 2000


def _embed_body(x_ref, wf_ref, bf_ref, ws_ref, wd_ref, brf_ref,
                h0_ref, a_ref, b_ref):
    h0 = jnp.dot(x_ref[...], wf_ref[...], preferred_element_type=jnp.float32)
    h0 = h0 + bf_ref[...]
    h0_ref[...] = h0
    a_ref[...] = jnp.dot(h0, ws_ref[...], preferred_element_type=jnp.float32)
    b_ref[...] = jnp.dot(h0, wd_ref[...],
                         preferred_element_type=jnp.float32) + brf_ref[...]


def _embed(x, wf, bf, ws, wd, brf):
    grid = N // BN
    return pl.pallas_call(
        _embed_body,
        grid=(grid,),
        in_specs=[
            pl.BlockSpec((BN, IN_DIM), lambda i: (i, 0)),
            pl.BlockSpec((IN_DIM, H), lambda i: (0, 0)),
            pl.BlockSpec((1, H), lambda i: (0, 0)),
            pl.BlockSpec((H, R * H), lambda i: (0, 0)),
            pl.BlockSpec((H, R * H), lambda i: (0, 0)),
            pl.BlockSpec((1, R * H), lambda i: (0, 0)),
        ],
        out_specs=(
            pl.BlockSpec((BN, H), lambda i: (i, 0)),
            pl.BlockSpec((BN, R * H), lambda i: (i, 0)),
            pl.BlockSpec((BN, R * H), lambda i: (i, 0)),
        ),
        out_shape=(
            jax.ShapeDtypeStruct((N, H), jnp.float32),
            jax.ShapeDtypeStruct((N, R * H), jnp.float32),
            jax.ShapeDtypeStruct((N, R * H), jnp.float32),
        ),
    )(x, wf, bf, ws, wd, brf)


def _sb(s4, b_ref):
    acc = s4[:, 0:1] * b_ref[:, 0 * H:1 * H]
    for r in range(1, R):
        acc = acc + s4[:, r:r + 1] * b_ref[:, r * H:(r + 1) * H]
    return acc


def _mid_body(p_ref, s_ref, b1_ref, ws_ref, wd_ref, brf_ref,
              h_ref, a_ref, b_ref):
    s4 = s_ref[0] + s_ref[1]
    h = p_ref[0] + p_ref[1] + _sb(s4, b1_ref)
    h_ref[...] = h
    a_ref[...] = jnp.dot(h, ws_ref[...], preferred_element_type=jnp.float32)
    b_ref[...] = jnp.dot(h, wd_ref[...],
                         preferred_element_type=jnp.float32) + brf_ref[...]


def _mid(p, s, b1, ws, wd, brf):
    grid = N // BN
    return pl.pallas_call(
        _mid_body,
        grid=(grid,),
        in_specs=[
            pl.BlockSpec((2, BN, H), lambda i: (0, i, 0)),
            pl.BlockSpec((2, BN, SW), lambda i: (0, i, 0)),
            pl.BlockSpec((BN, R * H), lambda i: (i, 0)),
            pl.BlockSpec((H, R * H), lambda i: (0, 0)),
            pl.BlockSpec((H, R * H), lambda i: (0, 0)),
            pl.BlockSpec((1, R * H), lambda i: (0, 0)),
        ],
        out_specs=(
            pl.BlockSpec((BN, H), lambda i: (i, 0)),
            pl.BlockSpec((BN, R * H), lambda i: (i, 0)),
            pl.BlockSpec((BN, R * H), lambda i: (i, 0)),
        ),
        out_shape=(
            jax.ShapeDtypeStruct((N, H), jnp.float32),
            jax.ShapeDtypeStruct((N, R * H), jnp.float32),
            jax.ShapeDtypeStruct((N, R * H), jnp.float32),
        ),
    )(p, s, b1, ws, wd, brf)


def _lrelu(t):
    return jnp.where(t > 0, t, 0.01 * t)


def _final_body(p2_ref, s_ref, b2_ref, h1_ref, h0_ref, wo0_ref, bo0_ref,
                wo1_ref, bo1_ref, wo2_ref, bo2_ref, out_ref):
    s4 = s_ref[0] + s_ref[1]
    h2 = p2_ref[0] + p2_ref[1] + _sb(s4, b2_ref)
    t2 = jnp.dot(h2, wo2_ref[...], preferred_element_type=jnp.float32) + bo2_ref[...]
    t1 = jnp.dot(h1_ref[...], wo1_ref[...],
                 preferred_element_type=jnp.float32) + bo1_ref[...]
    t0 = jnp.dot(h0_ref[...], wo0_ref[...],
                 preferred_element_type=jnp.float32) + bo0_ref[...]
    out_ref[...] = _lrelu(t2) + _lrelu(t1) + _lrelu(t0)


def _final(p2, s, b2, h1, h0, wo0, bo0, wo1, bo1, wo2, bo2):
    grid = N // BN
    wspec = pl.BlockSpec((H, OUT_DIM), lambda i: (0, 0))
    bspec = pl.BlockSpec((1, OUT_DIM), lambda i: (0, 0))
    return pl.pallas_call(
        _final_body,
        grid=(grid,),
        in_specs=[
            pl.BlockSpec((2, BN, H), lambda i: (0, i, 0)),
            pl.BlockSpec((2, BN, SW), lambda i: (0, i, 0)),
            pl.BlockSpec((BN, R * H), lambda i: (i, 0)),
            pl.BlockSpec((BN, H), lambda i: (i, 0)),
            pl.BlockSpec((BN, H), lambda i: (i, 0)),
            wspec, bspec, wspec, bspec, wspec, bspec,
        ],
        out_specs=pl.BlockSpec((BN, OUT_DIM), lambda i: (i, 0)),
        out_shape=jax.ShapeDtypeStruct((N, OUT_DIM), jnp.float32),
    )(p2, s, b2, h1, h0, wo0, bo0, wo1, bo1, wo2, bo2)


def _sc_layer_body_factory(with_s):
    def body(*refs):
        if with_s:
            (a_hbm, idxa_hbm, dst_hbm, typ_hbm, w_hbm, out_hbm, sout_hbm,
             idxa_v, dst_v, typ_v, w_v, a0, a1, o0, o1, s0b, s1b,
             zbuf, zbuf16, acc, sacc,
             sga0, sga1, ss0, ss1, sss0, sss1) = refs
        else:
            (a_hbm, idxa_hbm, dst_hbm, w_hbm, out_hbm,
             idxa_v, dst_v, w_v, a0, a1, o0, o1,
             zbuf, acc,
             sga0, sga1, ss0, ss1) = refs

        cid = lax.axis_index("c")
        sid = lax.axis_index("s")
        ng = jnp.where(cid == 0, GPT0, GPT1)
        gbase = jnp.where(cid == 0, sid * GPT0, 16 * GPT0 + sid * GPT1)
        sbase = jnp.minimum(gbase, G - GPTMAX)
        off = gbase - sbase
        pltpu.sync_copy(idxa_hbm.at[pl.ds(sbase, GPTMAX)], idxa_v)
        pltpu.sync_copy(dst_hbm.at[pl.ds(sbase, GPTMAX)], dst_v)
        pltpu.sync_copy(w_hbm.at[pl.ds(sbase, GPTMAX)], w_v)
        if with_s:
            pltpu.sync_copy(typ_hbm.at[pl.ds(sbase, GPTMAX)], typ_v)

        def zb(i, c):
            zbuf[i, 0:16] = jnp.zeros((16,), jnp.float32)
            zbuf[i, 16:32] = jnp.zeros((16,), jnp.float32)
            if with_s:
                zbuf16[i, 0:16] = jnp.zeros((16,), jnp.float32)
            return c

        lax.fori_loop(0, ZCH, zb, 0)
        rbase = sid * ROWS_PER_TILE
        for j in range(ROWS_PER_TILE // ZCH):
            pltpu.sync_copy(zbuf.at[pl.ds(0, ZCH)],
                            acc.at[pl.ds(rbase + j * ZCH, ZCH)])
            if with_s:
                pltpu.sync_copy(zbuf16.at[pl.ds(0, ZCH)],
                                sacc.at[pl.ds(rbase + j * ZCH, ZCH)])
        plsc.subcore_barrier()

        abufs = (a0, a1)
        obufs = (o0, o1)
        sgas = (sga0, sga1)
        sss = (ss0, ss1)
        if with_s:
            sbufs = (s0b, s1b)
            ssss = (sss0, sss1)

        for p in range(2):
            pltpu.async_copy(a_hbm.at[idxa_v.at[off + p]], abufs[p], sgas[p])

        iota16 = lax.iota(jnp.int32, 16)

        def pair(k2, c):
            for p in range(2):
                k = k2 * 2 + p
                ab, ob = abufs[p], obufs[p]
                pltpu.make_async_copy(a_hbm.at[idxa_v.at[off + k]], ab,
                                      sgas[p]).wait()

                @pl.when(k2 > 0)
                def _():
                    pltpu.make_async_copy(ob, acc.at[dst_v.at[off + k]],
                                          sss[p]).wait()
                    if with_s:
                        pltpu.make_async_copy(sbufs[p],
                                              sacc.at[dst_v.at[off + k]],
                                              ssss[p]).wait()

                def ebody(j, cc):
                    wv16 = w_v[off + k, pl.ds(j * 16, 16)]
                    if with_s:
                        tv16 = typ_v[off + k, pl.ds(j * 16, 16)]
                    for ll in range(16):
                        i = j * 16 + ll
                        wv = wv16[ll]
                        ob[i, 0:16] = ab[i, 0:16] * wv
                        ob[i, 16:32] = ab[i, 16:32] * wv
                        if with_s:
                            sbufs[p][i, 0:16] = jnp.where(
                                iota16 == tv16[ll], wv, 0.0)
                    return cc

                lax.fori_loop(0, GROUP // 16, ebody, 0)

                @pl.when(k + 2 < ng)
                def _():
                    pltpu.async_copy(a_hbm.at[idxa_v.at[off + k + 2]], ab,
                                     sgas[p])

                pltpu.async_copy(ob, acc.at[dst_v.at[off + k]], sss[p],
                                 add=True)
                if with_s:
                    pltpu.async_copy(sbufs[p], sacc.at[dst_v.at[off + k]],
                                     ssss[p], add=True)
            return c

        lax.fori_loop(0, (ng + 1) // 2, pair, 0)
        for p in range(2):
            pltpu.make_async_copy(obufs[p], acc.at[dst_v.at[off + ng - 2 + p]],
                                  sss[p]).wait()
            if with_s:
                pltpu.make_async_copy(sbufs[p],
                                      sacc.at[dst_v.at[off + ng - 2 + p]],
                                      ssss[p]).wait()
        plsc.subcore_barrier()

        pltpu.sync_copy(acc.at[pl.ds(rbase, ROWS_PER_TILE)],
                        out_hbm.at[cid, pl.ds(rbase, ROWS_PER_TILE)])
        if with_s:
            pltpu.sync_copy(sacc.at[pl.ds(rbase, ROWS_PER_TILE)],
                            sout_hbm.at[cid, pl.ds(rbase, ROWS_PER_TILE)])

    return body


def _sc_layer1(a2d, idxa2, dst2, typ2, w2):
    mesh = plsc.VectorSubcoreMesh(core_axis_name="c", subcore_axis_name="s")
    kern = functools.partial(
        pl.kernel,
        mesh=mesh,
        compiler_params=pltpu.CompilerParams(use_tc_tiling_on_sc=False),
        out_type=(
            jax.ShapeDtypeStruct((2, NP, H), jnp.float32),
            jax.ShapeDtypeStruct((2, NP, SW), jnp.float32),
        ),
        scratch_types=[
            pltpu.VMEM((GPTMAX, GROUP), jnp.int32),
            pltpu.VMEM((GPTMAX, GROUP), jnp.int32),
            pltpu.VMEM((GPTMAX, GROUP), jnp.int32),
            pltpu.VMEM((GPTMAX, GROUP), jnp.float32),
            pltpu.VMEM((GROUP, H), jnp.float32),
            pltpu.VMEM((GROUP, H), jnp.float32),
            pltpu.VMEM((GROUP, H), jnp.float32),
            pltpu.VMEM((GROUP, H), jnp.float32),
            pltpu.VMEM((GROUP, SW), jnp.float32),
            pltpu.VMEM((GROUP, SW), jnp.float32),
            pltpu.VMEM((ZCH, H), jnp.float32),
            pltpu.VMEM((ZCH, SW), jnp.float32),
            pltpu.VMEM_SHARED((NP, H), jnp.float32),
            pltpu.VMEM_SHARED((NP, SW), jnp.float32),
            pltpu.SemaphoreType.DMA,
            pltpu.SemaphoreType.DMA,
            pltpu.SemaphoreType.DMA,
            pltpu.SemaphoreType.DMA,
            pltpu.SemaphoreType.DMA,
            pltpu.SemaphoreType.DMA,
        ],
    )(_sc_layer_body_factory(True))
    return kern(a2d, idxa2, dst2, typ2, w2)


def _sc_layer2(a2d, idxa2, dst2, w2):
    mesh = plsc.VectorSubcoreMesh(core_axis_name="c", subcore_axis_name="s")
    kern = functools.partial(
        pl.kernel,
        mesh=mesh,
        compiler_params=pltpu.CompilerParams(use_tc_tiling_on_sc=False),
        out_type=jax.ShapeDtypeStruct((2, NP, H), jnp.float32),
        scratch_types=[
            pltpu.VMEM((GPTMAX, GROUP), jnp.int32),
            pltpu.VMEM((GPTMAX, GROUP), jnp.int32),
            pltpu.VMEM((GPTMAX, GROUP), jnp.float32),
            pltpu.VMEM((GROUP, H), jnp.float32),
            pltpu.VMEM((GROUP, H), jnp.float32),
            pltpu.VMEM((GROUP, H), jnp.float32),
            pltpu.VMEM((GROUP, H), jnp.float32),
            pltpu.VMEM((ZCH, H), jnp.float32),
            pltpu.VMEM_SHARED((NP, H), jnp.float32),
            pltpu.SemaphoreType.DMA,
            pltpu.SemaphoreType.DMA,
            pltpu.SemaphoreType.DMA,
            pltpu.SemaphoreType.DMA,
        ],
    )(_sc_layer_body_factory(False))
    return kern(a2d, idxa2, dst2, w2)


def kernel(x, edge_time, lambda_sym, beta, Wf, bf, Wr1, br1, Wr2, br2,
           Wo0, bo0, Wo1, bo1, Wo2, bo2, edge_index, edge_type):
    ws1 = jnp.transpose(Wr1[:, :H, :], (1, 0, 2)).reshape(H, R * H)
    wd1 = jnp.transpose(Wr1[:, H:, :], (1, 0, 2)).reshape(H, R * H)
    ws2 = jnp.transpose(Wr2[:, :H, :], (1, 0, 2)).reshape(H, R * H)
    wd2 = jnp.transpose(Wr2[:, H:, :], (1, 0, 2)).reshape(H, R * H)
    brf1 = br1.reshape(1, R * H)
    brf2 = br2.reshape(1, R * H)
    bfr = bf.reshape(1, H)
    bo0r = bo0.reshape(1, OUT_DIM)
    bo1r = bo1.reshape(1, OUT_DIM)
    bo2r = bo2.reshape(1, OUT_DIM)

    pad = EP - E
    et2 = jnp.pad(edge_time, (0, pad)).reshape(G, GROUP)
    src2 = jnp.pad(edge_index[0], (0, pad)).reshape(G, GROUP)
    dst2 = jnp.pad(edge_index[1], (0, pad)).reshape(G, GROUP)
    typ2 = jnp.pad(edge_type, (0, pad)).reshape(G, GROUP)

    w2, idxa2 = _edge_prep(et2, src2, typ2, lambda_sym, beta)

    h0, a1, b1 = _embed(x, Wf, bfr, ws1, wd1, brf1)
    p1, s1 = _sc_layer1(a1.reshape(NT, H), idxa2, dst2, typ2, w2)
    p1 = p1[:, :N, :]
    s = s1[:, :N, :]
    h1, a2, b2 = _mid(p1, s, b1, ws2, wd2, brf2)
    p2 = _sc_layer2(a2.reshape(NT, H), idxa2, dst2, w2)[:, :N, :]
    out = _final(p2, s, b2, h1, h0, Wo0, bo0r, Wo1, bo1r, Wo2, bo2r)
    return out


# speedup vs baseline: 1.0916x; 1.0138x over previous
import functools

import jax
import jax.numpy as jnp
from jax import lax
from jax.experimental import pallas as pl
from jax.experimental.pallas import tpu as pltpu
from jax.experimental.pallas import tpu_sc as plsc

N = 10000
E = 320000
IN_DIM = 128
H = 32
OUT_DIM = 128
R = 4

GROUP = 128
NW = 32
G = 2560
EP = G * GROUP
GPT = G // NW
GPT0 = 120
GPT1 = 40
GPTMAX = max(GPT0, GPT1)
NP = 10240
ROWS_PER_TILE = NP // 16
ZCH = 128
NT = N * R


def _edge_prep_body(et_ref, src_ref, dst_ref, typ_ref, lam_ref, beta_ref,
                    w_ref, ia_ref, ib_ref):
    lam = lam_ref[0, 0]
    beta = beta_ref[0, 0]
    valid = lax.broadcasted_iota(jnp.int32, (G, GROUP), 0) < (E // GROUP)
    w = lam * jnp.exp(-beta * jnp.abs(et_ref[...]))
    w_ref[...] = jnp.where(valid, w, 0.0)
    typ = typ_ref[...]
    ia_ref[...] = src_ref[...] * R + typ
    ib_ref[...] = dst_ref[...] * R + typ


def _edge_prep(et2, src2, dst2, typ2, lam, beta):
    return pl.pallas_call(
        _edge_prep_body,
        out_shape=(
            jax.ShapeDtypeStruct((G, GROUP), jnp.float32),
            jax.ShapeDtypeStruct((G, GROUP), jnp.int32),
            jax.ShapeDtypeStruct((G, GROUP), jnp.int32),
        ),
    )(et2, src2, dst2, typ2, lam, beta)


BN = 2000


def _embed_body(x_ref, wf_ref, bf_ref, ws_ref, wd_ref, brf_ref,
                h0_ref, a_ref, b_ref):
    h0 = jnp.dot(x_ref[...], wf_ref[...], preferred_element_type=jnp.float32)
    h0 = h0 + bf_ref[...]
    h0_ref[...] = h0
    a_ref[...] = jnp.dot(h0, ws_ref[...], preferred_element_type=jnp.float32)
    b_ref[...] = jnp.dot(h0, wd_ref[...],
                         preferred_element_type=jnp.float32) + brf_ref[...]


def _embed(x, wf, bf, ws, wd, brf):
    grid = N // BN
    return pl.pallas_call(
        _embed_body,
        grid=(grid,),
        in_specs=[
            pl.BlockSpec((BN, IN_DIM), lambda i: (i, 0)),
            pl.BlockSpec((IN_DIM, H), lambda i: (0, 0)),
            pl.BlockSpec((1, H), lambda i: (0, 0)),
            pl.BlockSpec((H, R * H), lambda i: (0, 0)),
            pl.BlockSpec((H, R * H), lambda i: (0, 0)),
            pl.BlockSpec((1, R * H), lambda i: (0, 0)),
        ],
        out_specs=(
            pl.BlockSpec((BN, H), lambda i: (i, 0)),
            pl.BlockSpec((BN, R * H), lambda i: (i, 0)),
            pl.BlockSpec((BN, R * H), lambda i: (i, 0)),
        ),
        out_shape=(
            jax.ShapeDtypeStruct((N, H), jnp.float32),
            jax.ShapeDtypeStruct((N, R * H), jnp.float32),
            jax.ShapeDtypeStruct((N, R * H), jnp.float32),
        ),
    )(x, wf, bf, ws, wd, brf)


def _mid_body(p_ref, ws_ref, wd_ref, brf_ref, h_ref, a_ref, b_ref):
    h = p_ref[0] + p_ref[1]
    h_ref[...] = h
    a_ref[...] = jnp.dot(h, ws_ref[...], preferred_element_type=jnp.float32)
    b_ref[...] = jnp.dot(h, wd_ref[...],
                         preferred_element_type=jnp.float32) + brf_ref[...]


def _mid(p, ws, wd, brf):
    grid = N // BN
    return pl.pallas_call(
        _mid_body,
        grid=(grid,),
        in_specs=[
            pl.BlockSpec((2, BN, H), lambda i: (0, i, 0)),
            pl.BlockSpec((H, R * H), lambda i: (0, 0)),
            pl.BlockSpec((H, R * H), lambda i: (0, 0)),
            pl.BlockSpec((1, R * H), lambda i: (0, 0)),
        ],
        out_specs=(
            pl.BlockSpec((BN, H), lambda i: (i, 0)),
            pl.BlockSpec((BN, R * H), lambda i: (i, 0)),
            pl.BlockSpec((BN, R * H), lambda i: (i, 0)),
        ),
        out_shape=(
            jax.ShapeDtypeStruct((N, H), jnp.float32),
            jax.ShapeDtypeStruct((N, R * H), jnp.float32),
            jax.ShapeDtypeStruct((N, R * H), jnp.float32),
        ),
    )(p, ws, wd, brf)


def _lrelu(t):
    return jnp.where(t > 0, t, 0.01 * t)


def _final_body(p2_ref, h1_ref, h0_ref, wo0_ref, bo0_ref, wo1_ref, bo1_ref,
                wo2_ref, bo2_ref, out_ref):
    h2 = p2_ref[0] + p2_ref[1]
    t2 = jnp.dot(h2, wo2_ref[...], preferred_element_type=jnp.float32) + bo2_ref[...]
    t1 = jnp.dot(h1_ref[...], wo1_ref[...],
                 preferred_element_type=jnp.float32) + bo1_ref[...]
    t0 = jnp.dot(h0_ref[...], wo0_ref[...],
                 preferred_element_type=jnp.float32) + bo0_ref[...]
    out_ref[...] = _lrelu(t2) + _lrelu(t1) + _lrelu(t0)


def _final(p2, h1, h0, wo0, bo0, wo1, bo1, wo2, bo2):
    grid = N // BN
    wspec = pl.BlockSpec((H, OUT_DIM), lambda i: (0, 0))
    bspec = pl.BlockSpec((1, OUT_DIM), lambda i: (0, 0))
    return pl.pallas_call(
        _final_body,
        grid=(grid,),
        in_specs=[
            pl.BlockSpec((2, BN, H), lambda i: (0, i, 0)),
            pl.BlockSpec((BN, H), lambda i: (i, 0)),
            pl.BlockSpec((BN, H), lambda i: (i, 0)),
            wspec, bspec, wspec, bspec, wspec, bspec,
        ],
        out_specs=pl.BlockSpec((BN, OUT_DIM), lambda i: (i, 0)),
        out_shape=jax.ShapeDtypeStruct((N, OUT_DIM), jnp.float32),
    )(p2, h1, h0, wo0, bo0, wo1, bo1, wo2, bo2)


def _sc_layer_body(a_hbm, b_hbm, idxa_hbm, idxb_hbm, dst_hbm, w_hbm, out_hbm,
                   idxa_v, idxb_v, dst_v, w_v, a0, a1, b0, b1, o0, o1,
                   zbuf, acc, sga0, sga1, sgb0, sgb1, ss0, ss1):
    cid = lax.axis_index("c")
    sid = lax.axis_index("s")
    ng = jnp.where(cid == 0, GPT0, GPT1)
    gbase = jnp.where(cid == 0, sid * GPT0, 16 * GPT0 + sid * GPT1)

    sbase = jnp.minimum(gbase, G - GPTMAX)
    off = gbase - sbase
    pltpu.sync_copy(idxa_hbm.at[pl.ds(sbase, GPTMAX)], idxa_v)
    pltpu.sync_copy(idxb_hbm.at[pl.ds(sbase, GPTMAX)], idxb_v)
    pltpu.sync_copy(dst_hbm.at[pl.ds(sbase, GPTMAX)], dst_v)
    pltpu.sync_copy(w_hbm.at[pl.ds(sbase, GPTMAX)], w_v)

    def zb(i, c):
        zbuf[i, 0:16] = jnp.zeros((16,), jnp.float32)
        zbuf[i, 16:32] = jnp.zeros((16,), jnp.float32)
        return c

    lax.fori_loop(0, ZCH, zb, 0)
    rbase = sid * ROWS_PER_TILE
    for j in range(ROWS_PER_TILE // ZCH):
        pltpu.sync_copy(zbuf.at[pl.ds(0, ZCH)],
                        acc.at[pl.ds(rbase + j * ZCH, ZCH)])
    plsc.subcore_barrier()

    abufs = (a0, a1)
    bbufs = (b0, b1)
    obufs = (o0, o1)
    sgas = (sga0, sga1)
    sgbs = (sgb0, sgb1)
    sss = (ss0, ss1)

    for p in range(2):
        pltpu.async_copy(a_hbm.at[idxa_v.at[off + p]], abufs[p], sgas[p])
        pltpu.async_copy(b_hbm.at[idxb_v.at[off + p]], bbufs[p], sgbs[p])

    def pair(k2, c):
        for p in range(2):
            k = k2 * 2 + p
            ab, bb, ob = abufs[p], bbufs[p], obufs[p]
            pltpu.make_async_copy(a_hbm.at[idxa_v.at[off + k]], ab, sgas[p]).wait()
            pltpu.make_async_copy(b_hbm.at[idxb_v.at[off + k]], bb, sgbs[p]).wait()

            @pl.when(k2 > 0)
            def _():
                pltpu.make_async_copy(ob, acc.at[dst_v.at[off + k]], sss[p]).wait()

            def ebody(j, cc):
                wv16 = w_v[off + k, pl.ds(j * 16, 16)]
                for ll in range(16):
                    i = j * 16 + ll
                    wv = wv16[ll]
                    ob[i, 0:16] = (ab[i, 0:16] + bb[i, 0:16]) * wv
                    ob[i, 16:32] = (ab[i, 16:32] + bb[i, 16:32]) * wv
                return cc

            lax.fori_loop(0, GROUP // 16, ebody, 0)

            @pl.when(k + 2 < ng)
            def _():
                pltpu.async_copy(a_hbm.at[idxa_v.at[off + k + 2]], ab, sgas[p])
                pltpu.async_copy(b_hbm.at[idxb_v.at[off + k + 2]], bb, sgbs[p])

            pltpu.async_copy(ob, acc.at[dst_v.at[off + k]], sss[p], add=True)
        return c

    lax.fori_loop(0, ng // 2, pair, 0)
    for p in range(2):
        pltpu.make_async_copy(obufs[p], acc.at[dst_v.at[off + ng - 2 + p]],
                              sss[p]).wait()
    plsc.subcore_barrier()

    pltpu.sync_copy(acc.at[pl.ds(rbase, ROWS_PER_TILE)],
                    out_hbm.at[cid, pl.ds(rbase, ROWS_PER_TILE)])


def _sc_layer(a2d, b2d, idxa2, idxb2, dst2, w2):
    mesh = plsc.VectorSubcoreMesh(core_axis_name="c", subcore_axis_name="s")
    kern = functools.partial(
        pl.kernel,
        mesh=mesh,
        compiler_params=pltpu.CompilerParams(use_tc_tiling_on_sc=False),
        out_type=jax.ShapeDtypeStruct((2, NP, H), jnp.float32),
        scratch_types=[
            pltpu.VMEM((GPTMAX, GROUP), jnp.int32),
            pltpu.VMEM((GPTMAX, GROUP), jnp.int32),
            pltpu.VMEM((GPTMAX, GROUP), jnp.int32),
            pltpu.VMEM((GPTMAX, GROUP), jnp.float32),
            pltpu.VMEM((GROUP, H), jnp.float32),
            pltpu.VMEM((GROUP, H), jnp.float32),
            pltpu.VMEM((GROUP, H), jnp.float32),
            pltpu.VMEM((GROUP, H), jnp.float32),
            pltpu.VMEM((GROUP, H), jnp.float32),
            pltpu.VMEM((GROUP, H), jnp.float32),
            pltpu.VMEM((ZCH, H), jnp.float32),
            pltpu.VMEM_SHARED((NP, H), jnp.float32),
            pltpu.SemaphoreType.DMA,
            pltpu.SemaphoreType.DMA,
            pltpu.SemaphoreType.DMA,
            pltpu.SemaphoreType.DMA,
            pltpu.SemaphoreType.DMA,
            pltpu.SemaphoreType.DMA,
        ],
    )(_sc_layer_body)
    return kern(a2d, b2d, idxa2, idxb2, dst2, w2)


def kernel(x, edge_time, lambda_sym, beta, Wf, bf, Wr1, br1, Wr2, br2,
           Wo0, bo0, Wo1, bo1, Wo2, bo2, edge_index, edge_type):
    ws1 = jnp.transpose(Wr1[:, :H, :], (1, 0, 2)).reshape(H, R * H)
    wd1 = jnp.transpose(Wr1[:, H:, :], (1, 0, 2)).reshape(H, R * H)
    ws2 = jnp.transpose(Wr2[:, :H, :], (1, 0, 2)).reshape(H, R * H)
    wd2 = jnp.transpose(Wr2[:, H:, :], (1, 0, 2)).reshape(H, R * H)
    brf1 = br1.reshape(1, R * H)
    brf2 = br2.reshape(1, R * H)
    bfr = bf.reshape(1, H)
    bo0r = bo0.reshape(1, OUT_DIM)
    bo1r = bo1.reshape(1, OUT_DIM)
    bo2r = bo2.reshape(1, OUT_DIM)

    pad = EP - E
    et2 = jnp.pad(edge_time, (0, pad)).reshape(G, GROUP)
    src2 = jnp.pad(edge_index[0], (0, pad)).reshape(G, GROUP)
    dst2 = jnp.pad(edge_index[1], (0, pad)).reshape(G, GROUP)
    typ2 = jnp.pad(edge_type, (0, pad)).reshape(G, GROUP)

    w2, idxa2, idxb2 = _edge_prep(et2, src2, dst2, typ2, lambda_sym, beta)

    h0, a1, b1 = _embed(x, Wf, bfr, ws1, wd1, brf1)
    p1 = _sc_layer(a1.reshape(NT, H), b1.reshape(NT, H),
                   idxa2, idxb2, dst2, w2)[:, :N, :]
    h1, a2, b2 = _mid(p1, ws2, wd2, brf2)
    p2 = _sc_layer(a2.reshape(NT, H), b2.reshape(NT, H),
                   idxa2, idxb2, dst2, w2)[:, :N, :]
    out = _final(p2, h1, h0, Wo0, bo0r, Wo1, bo1r, Wo2, bo2r)
    return out
